# baseline (speedup 1.0000x reference)
import functools

import jax
import jax.numpy as jnp
import numpy as np
from jax import lax
from jax.experimental import pallas as pl
from jax.experimental.pallas import tpu as pltpu
from jax.experimental.pallas import tpu_sc as plsc

_B = 16384
_F = 26
_V = 100000
_D = 16

_NC = 2
_NS = 16
_NW = _NC * _NS
_N = _B * _F
_PER_W = _N // _NW
_CHUNK = 128
_NBUF = 2
_NSTEPS = _PER_W // (_CHUNK * _NBUF)


_CPF = _B // _CHUNK


def _gather_body(emb_hbm, lin16_hbm, xt_hbm, emb_out, lin_out,
                 idx_bufs, idx16_bufs, row_bufs, lin16_bufs, linval_bufs,
                 sems_e, sems_l):
    wid = lax.axis_index("s") * _NC + lax.axis_index("c")
    base_c = wid * _PER_W // _CHUNK
    lane_iota = lax.iota(jnp.int32, 16)

    def step(m, carry):
        cs = [base_c + m * _NBUF + b for b in range(_NBUF)]
        fs = [c // _CPF for c in cs]
        bs = [(c % _CPF) * _CHUNK for c in cs]
        for b in range(_NBUF):
            pltpu.sync_copy(xt_hbm.at[fs[b], pl.ds(bs[b], _CHUNK)],
                            idx_bufs[b])
            for g in range(_CHUNK // 16):
                sl = pl.ds(g * 16, 16)
                idx16_bufs[b][sl] = jnp.right_shift(idx_bufs[b][sl], 4)
        cps = []
        for b in range(_NBUF):
            cps.append(pltpu.async_copy(emb_hbm.at[fs[b]].at[idx_bufs[b]],
                                        row_bufs[b], sems_e[b]))
            cps.append(pltpu.async_copy(lin16_hbm.at[fs[b]].at[idx16_bufs[b]],
                                        lin16_bufs[b], sems_l[b]))
        for c in cps:
            c.wait()
        for b in range(_NBUF):
            for g in range(_CHUNK // 16):
                sl = pl.ds(g * 16, 16)
                rows = lane_iota + g * 16
                lanes = jnp.bitwise_and(idx_bufs[b][sl], 15)
                linval_bufs[b][sl] = plsc.load_gather(
                    lin16_bufs[b], [rows, lanes])
            pltpu.sync_copy(row_bufs[b],
                            emb_out.at[pl.ds(bs[b], _CHUNK),
                                       pl.ds(fs[b] * _D, _D)])
            pltpu.sync_copy(linval_bufs[b],
                            lin_out.at[fs[b], pl.ds(bs[b], _CHUNK)])
        return carry

    lax.fori_loop(0, _NSTEPS, step, 0)


@functools.lru_cache(maxsize=None)
def _make_sc_gather():
    return functools.partial(
        pl.kernel,
        mesh=plsc.VectorSubcoreMesh(core_axis_name="c", subcore_axis_name="s",
                                    num_cores=_NC, num_subcores=_NS),
        out_type=[
            jax.ShapeDtypeStruct((_B, _F * _D), jnp.float32),
            jax.ShapeDtypeStruct((_F, _B), jnp.float32),
        ],
        scratch_types=[
            [pltpu.VMEM((_CHUNK,), jnp.int32) for _ in range(_NBUF)],
            [pltpu.VMEM((_CHUNK,), jnp.int32) for _ in range(_NBUF)],
            [pltpu.VMEM((_CHUNK, _D), jnp.float32) for _ in range(_NBUF)],
            [pltpu.VMEM((_CHUNK, 16), jnp.float32) for _ in range(_NBUF)],
            [pltpu.VMEM((_CHUNK,), jnp.float32) for _ in range(_NBUF)],
            [pltpu.SemaphoreType.DMA for _ in range(_NBUF)],
            [pltpu.SemaphoreType.DMA for _ in range(_NBUF)],
        ],
        compiler_params=pltpu.CompilerParams(use_tc_tiling_on_sc=False,
                                             needs_layout_passes=False),
    )(_gather_body)


def _mlp_body(emb_ref, xd_ref, lin_ref, w0e_ref, w0d_ref, b0_ref,
              w1_ref, b1_ref, wo_ref, bo_ref, s_ref, out_ref):
    dot = functools.partial(jnp.dot, preferred_element_type=jnp.float32,
                            precision=lax.Precision.HIGHEST)
    emb = emb_ref[...]
    h = dot(emb, w0e_ref[...])
    h = h + dot(xd_ref[...], w0d_ref[...])
    h = jnp.maximum(h + b0_ref[...], 0.0)
    h = jnp.maximum(dot(h, w1_ref[...]) + b1_ref[...], 0.0)
    dnn = dot(h, wo_ref[...]) + bo_ref[...]
    s = dot(emb, s_ref[...])
    fm = 0.5 * (jnp.sum(s * s, axis=1, keepdims=True)
                - jnp.sum(emb * emb, axis=1, keepdims=True))
    lin_logit = jnp.sum(lin_ref[...], axis=1, keepdims=True)
    logit = dnn + fm + lin_logit
    out_ref[...] = 1.0 / (1.0 + jnp.exp(-logit))


def _tc_mlp(emb2, x_dense, lin2, w0e, w0d, b0, w1, b1, wo, bo, smat):
    bm = 1024
    grid = (_B // bm,)
    return pl.pallas_call(
        _mlp_body,
        grid=grid,
        in_specs=[
            pl.BlockSpec((bm, _F * _D), lambda i: (i, 0)),
            pl.BlockSpec((bm, x_dense.shape[1]), lambda i: (i, 0)),
            pl.BlockSpec((bm, _F), lambda i: (i, 0)),
            pl.BlockSpec(w0e.shape, lambda i: (0, 0)),
            pl.BlockSpec(w0d.shape, lambda i: (0, 0)),
            pl.BlockSpec(b0.shape, lambda i: (0,)),
            pl.BlockSpec(w1.shape, lambda i: (0, 0)),
            pl.BlockSpec(b1.shape, lambda i: (0,)),
            pl.BlockSpec(wo.shape, lambda i: (0, 0)),
            pl.BlockSpec(bo.shape, lambda i: (0,)),
            pl.BlockSpec(smat.shape, lambda i: (0, 0)),
        ],
        out_specs=pl.BlockSpec((bm, 1), lambda i: (i, 0)),
        out_shape=jax.ShapeDtypeStruct((_B, 1), jnp.float32),
    )(emb2, x_dense, lin2, w0e, w0d, b0, w1, b1, wo, bo, smat)


def kernel(x_sparse, x_dense, emb_tables, lin_tables, W0, b0, W1, b1, Wo, bo):
    nf = emb_tables.shape[0]
    v = emb_tables.shape[1]
    d = emb_tables.shape[2]
    xt = x_sparse.astype(jnp.int32).T
    lin16 = lin_tables.reshape(nf, v // 16, 16)
    emb2, lin_t = _make_sc_gather()(emb_tables, lin16, xt)
    lin2 = lin_t.T
    w0e = W0[:nf * d]
    w0d = W0[nf * d:]
    smat = jnp.tile(jnp.eye(d, dtype=jnp.float32), (nf, 1))
    out = _tc_mlp(emb2, x_dense, lin2, w0e, w0d, b0, W1, b1, Wo, bo, smat)
    return out.reshape(_B)


# speedup vs baseline: 1.3619x; 1.3619x over previous
import functools

import jax
import jax.numpy as jnp
import numpy as np
from jax import lax
from jax.experimental import pallas as pl
from jax.experimental.pallas import tpu as pltpu
from jax.experimental.pallas import tpu_sc as plsc

_B = 16384
_F = 26
_V = 100000
_D = 16

_NC = 2
_NS = 16
_NW = _NC * _NS
_N = _B * _F
_PER_W = _N // _NW
_CHUNK = 128
_NBUF = 2
_NSTEPS = _PER_W // (_CHUNK * _NBUF)
_CPF = _B // _CHUNK


def _gather_body(embt_hbm, lin16_hbm, xt_hbm, emb_out, lin_out,
                 idx_bufs, idx16_bufs, row_bufs, lin16_bufs,
                 col_bufs, linval_bufs, sems_e, sems_l):
    wid = lax.axis_index("s") * _NC + lax.axis_index("c")
    base_c = wid * _PER_W // _CHUNK
    lane_iota = lax.iota(jnp.int32, 16)

    def step(m, carry):
        cs = [base_c + m * _NBUF + b for b in range(_NBUF)]
        fs = [c // _CPF for c in cs]
        bs = [(c % _CPF) * _CHUNK for c in cs]
        for b in range(_NBUF):
            pltpu.sync_copy(xt_hbm.at[fs[b], pl.ds(bs[b], _CHUNK)],
                            idx_bufs[b])
            for g in range(_CHUNK // 16):
                sl = pl.ds(g * 16, 16)
                idx16_bufs[b][sl] = jnp.right_shift(idx_bufs[b][sl], 4)
        cps = []
        for b in range(_NBUF):
            for d in range(_D):
                cps.append(pltpu.async_copy(
                    embt_hbm.at[fs[b], d].at[idx16_bufs[b]],
                    row_bufs[b][d], sems_e[b]))
            cps.append(pltpu.async_copy(lin16_hbm.at[fs[b]].at[idx16_bufs[b]],
                                        lin16_bufs[b], sems_l[b]))
        for c in cps:
            c.wait()
        for b in range(_NBUF):
            lanes = [jnp.bitwise_and(idx_bufs[b][pl.ds(g * 16, 16)], 15)
                     for g in range(_CHUNK // 16)]
            rows = [lane_iota + g * 16 for g in range(_CHUNK // 16)]
            for g in range(_CHUNK // 16):
                sl = pl.ds(g * 16, 16)
                for d in range(_D):
                    col_bufs[b][d][sl] = plsc.load_gather(
                        row_bufs[b][d], [rows[g], lanes[g]])
                linval_bufs[b][sl] = plsc.load_gather(
                    lin16_bufs[b], [rows[g], lanes[g]])
            for d in range(_D):
                pltpu.sync_copy(col_bufs[b][d],
                                emb_out.at[fs[b] * _D + d,
                                           pl.ds(bs[b], _CHUNK)])
            pltpu.sync_copy(linval_bufs[b],
                            lin_out.at[fs[b], pl.ds(bs[b], _CHUNK)])
        return carry

    lax.fori_loop(0, _NSTEPS, step, 0)


@functools.lru_cache(maxsize=None)
def _make_sc_gather():
    return functools.partial(
        pl.kernel,
        mesh=plsc.VectorSubcoreMesh(core_axis_name="c", subcore_axis_name="s",
                                    num_cores=_NC, num_subcores=_NS),
        out_type=[
            jax.ShapeDtypeStruct((_F * _D, _B), jnp.float32),
            jax.ShapeDtypeStruct((_F, _B), jnp.float32),
        ],
        scratch_types=[
            [pltpu.VMEM((_CHUNK,), jnp.int32) for _ in range(_NBUF)],
            [pltpu.VMEM((_CHUNK,), jnp.int32) for _ in range(_NBUF)],
            [[pltpu.VMEM((_CHUNK, 16), jnp.float32) for _ in range(_D)]
             for _ in range(_NBUF)],
            [pltpu.VMEM((_CHUNK, 16), jnp.float32) for _ in range(_NBUF)],
            [[pltpu.VMEM((_CHUNK,), jnp.float32) for _ in range(_D)]
             for _ in range(_NBUF)],
            [pltpu.VMEM((_CHUNK,), jnp.float32) for _ in range(_NBUF)],
            [pltpu.SemaphoreType.DMA for _ in range(_NBUF)],
            [pltpu.SemaphoreType.DMA for _ in range(_NBUF)],
        ],
        compiler_params=pltpu.CompilerParams(use_tc_tiling_on_sc=False,
                                             needs_layout_passes=False),
    )(_gather_body)


def _mlp_body(embt_ref, xd_ref, lint_ref, w0e_ref, w0d_ref, b0_ref,
              w1_ref, b1_ref, wo_ref, bo_ref, st_ref, out_ref):
    dotn = functools.partial(jnp.dot, preferred_element_type=jnp.float32,
                             precision=lax.Precision.HIGHEST)

    def dott(a_t, w):
        return lax.dot_general(a_t, w, (((0,), (0,)), ((), ())),
                               preferred_element_type=jnp.float32,
                               precision=lax.Precision.HIGHEST)

    embt = embt_ref[...]
    h = dott(embt, w0e_ref[...])
    h = h + dotn(xd_ref[...], w0d_ref[...])
    h = jnp.maximum(h + b0_ref[...], 0.0)
    h = jnp.maximum(dotn(h, w1_ref[...]) + b1_ref[...], 0.0)
    dnn = dotn(h, wo_ref[...]) + bo_ref[...]
    st = dott(embt, st_ref[...])
    fm = 0.5 * (jnp.sum(st * st, axis=1, keepdims=True)
                - jnp.sum(embt * embt, axis=0)[:, None])
    lin_logit = jnp.sum(lint_ref[...], axis=0)[:, None]
    logit = dnn + fm + lin_logit
    out_ref[...] = 1.0 / (1.0 + jnp.exp(-logit))


def _tc_mlp(embt, x_dense, lint, w0e, w0d, b0, w1, b1, wo, bo, smat):
    bm = 1024
    grid = (_B // bm,)
    return pl.pallas_call(
        _mlp_body,
        grid=grid,
        in_specs=[
            pl.BlockSpec((_F * _D, bm), lambda i: (0, i)),
            pl.BlockSpec((bm, x_dense.shape[1]), lambda i: (i, 0)),
            pl.BlockSpec((_F, bm), lambda i: (0, i)),
            pl.BlockSpec(w0e.shape, lambda i: (0, 0)),
            pl.BlockSpec(w0d.shape, lambda i: (0, 0)),
            pl.BlockSpec(b0.shape, lambda i: (0,)),
            pl.BlockSpec(w1.shape, lambda i: (0, 0)),
            pl.BlockSpec(b1.shape, lambda i: (0,)),
            pl.BlockSpec(wo.shape, lambda i: (0, 0)),
            pl.BlockSpec(bo.shape, lambda i: (0,)),
            pl.BlockSpec(smat.shape, lambda i: (0, 0)),
        ],
        out_specs=pl.BlockSpec((bm, 1), lambda i: (i, 0)),
        out_shape=jax.ShapeDtypeStruct((_B, 1), jnp.float32),
    )(embt, x_dense, lint, w0e, w0d, b0, w1, b1, wo, bo, smat)


def kernel(x_sparse, x_dense, emb_tables, lin_tables, W0, b0, W1, b1, Wo, bo):
    nf = emb_tables.shape[0]
    v = emb_tables.shape[1]
    d = emb_tables.shape[2]
    xt = x_sparse.astype(jnp.int32).T
    embt = jnp.transpose(emb_tables, (0, 2, 1)).reshape(nf, d, v // 16, 16)
    lin16 = lin_tables.reshape(nf, v // 16, 16)
    embt_out, lin_t = _make_sc_gather()(embt, lin16, xt)
    w0e = W0[:nf * d]
    w0d = W0[nf * d:]
    smat = jnp.tile(jnp.eye(d, dtype=jnp.float32), (nf, 1))
    out = _tc_mlp(embt_out, x_dense, lin_t, w0e, w0d, b0, W1, b1, Wo, bo,
                  smat)
    return out.reshape(_B)


# speedup vs baseline: 1.4867x; 1.0916x over previous
import functools

import jax
import jax.numpy as jnp
import numpy as np
from jax import lax
from jax.experimental import pallas as pl
from jax.experimental.pallas import tpu as pltpu
from jax.experimental.pallas import tpu_sc as plsc

_B = 16384
_F = 26
_V = 100000
_D = 16

_NC = 2
_NS = 16
_NW = _NC * _NS
_N = _B * _F
_PER_W = _N // _NW
_CHUNK = 128
_NBUF = 2
_NSTEPS = _PER_W // (_CHUNK * _NBUF)
_CPF = _B // _CHUNK


def _gather_body(embt_hbm, lin16_hbm, xt_hbm, emb_out, lin_out,
                 idx_bufs, idx16_bufs, row_bufs, lin16_bufs,
                 col_bufs, linval_bufs, sems_e, sems_l):
    wid = lax.axis_index("s") * _NC + lax.axis_index("c")
    base_c = wid * _PER_W // _CHUNK
    lane_iota = lax.iota(jnp.int32, 16)

    def step(m, carry):
        cs = [base_c + m * _NBUF + b for b in range(_NBUF)]
        fs = [c // _CPF for c in cs]
        bs = [(c % _CPF) * _CHUNK for c in cs]
        for b in range(_NBUF):
            pltpu.sync_copy(xt_hbm.at[fs[b], pl.ds(bs[b], _CHUNK)],
                            idx_bufs[b])
            for g in range(_CHUNK // 16):
                sl = pl.ds(g * 16, 16)
                idx16_bufs[b][sl] = jnp.right_shift(idx_bufs[b][sl], 4)
        cps = []
        for b in range(_NBUF):
            for d in range(_D):
                cps.append(pltpu.async_copy(
                    embt_hbm.at[fs[b], d].at[idx16_bufs[b]],
                    row_bufs[b][d], sems_e[b]))
            cps.append(pltpu.async_copy(lin16_hbm.at[fs[b], 0].at[idx16_bufs[b]],
                                        lin16_bufs[b], sems_l[b]))
        for c in cps:
            c.wait()
        for b in range(_NBUF):
            lanes = [jnp.bitwise_and(idx_bufs[b][pl.ds(g * 16, 16)], 15)
                     for g in range(_CHUNK // 16)]
            rows = [lane_iota + g * 16 for g in range(_CHUNK // 16)]
            for g in range(_CHUNK // 16):
                sl = pl.ds(g * 16, 16)
                for d in range(_D):
                    col_bufs[b][d, sl] = plsc.load_gather(
                        row_bufs[b][d], [rows[g], lanes[g]])
                linval_bufs[b][sl] = plsc.load_gather(
                    lin16_bufs[b], [rows[g], lanes[g]])
            pltpu.sync_copy(col_bufs[b],
                            emb_out.at[pl.ds(fs[b] * _D, _D),
                                       pl.ds(bs[b], _CHUNK)])
            pltpu.sync_copy(linval_bufs[b],
                            lin_out.at[fs[b], pl.ds(bs[b], _CHUNK)])
        return carry

    lax.fori_loop(0, _NSTEPS, step, 0)


@functools.lru_cache(maxsize=None)
def _make_sc_gather():
    return functools.partial(
        pl.kernel,
        mesh=plsc.VectorSubcoreMesh(core_axis_name="c", subcore_axis_name="s",
                                    num_cores=_NC, num_subcores=_NS),
        out_type=[
            jax.ShapeDtypeStruct((_F * _D, _B), jnp.float32),
            jax.ShapeDtypeStruct((_F, _B), jnp.float32),
        ],
        scratch_types=[
            [pltpu.VMEM((_CHUNK,), jnp.int32) for _ in range(_NBUF)],
            [pltpu.VMEM((_CHUNK,), jnp.int32) for _ in range(_NBUF)],
            [[pltpu.VMEM((_CHUNK, 16), jnp.float32) for _ in range(_D)]
             for _ in range(_NBUF)],
            [pltpu.VMEM((_CHUNK, 16), jnp.float32) for _ in range(_NBUF)],
            [pltpu.VMEM((_D, _CHUNK), jnp.float32) for _ in range(_NBUF)],
            [pltpu.VMEM((_CHUNK,), jnp.float32) for _ in range(_NBUF)],
            [pltpu.SemaphoreType.DMA for _ in range(_NBUF)],
            [pltpu.SemaphoreType.DMA for _ in range(_NBUF)],
        ],
        compiler_params=pltpu.CompilerParams(use_tc_tiling_on_sc=False,
                                             needs_layout_passes=False),
    )(_gather_body)


def _mlp_body(embt_ref, xd_ref, lint_ref, w0e_ref, w0d_ref, b0_ref,
              w1_ref, b1_ref, wo_ref, bo_ref, st_ref, out_ref):
    dotn = functools.partial(jnp.dot, preferred_element_type=jnp.float32,
                             precision=lax.Precision.HIGHEST)

    def dott(a_t, w):
        return lax.dot_general(a_t, w, (((0,), (0,)), ((), ())),
                               preferred_element_type=jnp.float32,
                               precision=lax.Precision.HIGHEST)

    embt = embt_ref[...]
    h = dott(embt, w0e_ref[...])
    h = h + dotn(xd_ref[...], w0d_ref[...])
    h = jnp.maximum(h + b0_ref[...], 0.0)
    h = jnp.maximum(dotn(h, w1_ref[...]) + b1_ref[...], 0.0)
    dnn = dotn(h, wo_ref[...]) + bo_ref[...]
    st = dott(embt, st_ref[...])
    fm = 0.5 * (jnp.sum(st * st, axis=1, keepdims=True)
                - jnp.sum(embt * embt, axis=0)[:, None])
    lin_logit = jnp.sum(lint_ref[...], axis=0)[:, None]
    logit = dnn + fm + lin_logit
    out_ref[...] = 1.0 / (1.0 + jnp.exp(-logit))


def _tc_mlp(embt, x_dense, lint, w0e, w0d, b0, w1, b1, wo, bo, smat):
    bm = 1024
    grid = (_B // bm,)
    return pl.pallas_call(
        _mlp_body,
        grid=grid,
        in_specs=[
            pl.BlockSpec((_F * _D, bm), lambda i: (0, i)),
            pl.BlockSpec((bm, x_dense.shape[1]), lambda i: (i, 0)),
            pl.BlockSpec((_F, bm), lambda i: (0, i)),
            pl.BlockSpec(w0e.shape, lambda i: (0, 0)),
            pl.BlockSpec(w0d.shape, lambda i: (0, 0)),
            pl.BlockSpec(b0.shape, lambda i: (0,)),
            pl.BlockSpec(w1.shape, lambda i: (0, 0)),
            pl.BlockSpec(b1.shape, lambda i: (0,)),
            pl.BlockSpec(wo.shape, lambda i: (0, 0)),
            pl.BlockSpec(bo.shape, lambda i: (0,)),
            pl.BlockSpec(smat.shape, lambda i: (0, 0)),
        ],
        out_specs=pl.BlockSpec((bm, 1), lambda i: (i, 0)),
        out_shape=jax.ShapeDtypeStruct((_B, 1), jnp.float32),
    )(embt, x_dense, lint, w0e, w0d, b0, w1, b1, wo, bo, smat)


def kernel(x_sparse, x_dense, emb_tables, lin_tables, W0, b0, W1, b1, Wo, bo):
    nf = emb_tables.shape[0]
    v = emb_tables.shape[1]
    d = emb_tables.shape[2]
    xt = x_sparse.astype(jnp.int32).T
    embt = jnp.transpose(emb_tables, (0, 2, 1)).reshape(nf, d, v // 16, 16)
    lin16 = jnp.transpose(lin_tables, (0, 2, 1)).reshape(nf, 1, v // 16, 16)
    embt_out, lin_t = _make_sc_gather()(embt, lin16, xt)
    w0e = W0[:nf * d]
    w0d = W0[nf * d:]
    smat = jnp.tile(jnp.eye(d, dtype=jnp.float32), (nf, 1))
    out = _tc_mlp(embt_out, x_dense, lin_t, w0e, w0d, b0, W1, b1, Wo, bo,
                  smat)
    return out.reshape(_B)


# speedup vs baseline: 1.6244x; 1.0926x over previous
import functools

import jax
import jax.numpy as jnp
import numpy as np
from jax import lax
from jax.experimental import pallas as pl
from jax.experimental.pallas import tpu as pltpu
from jax.experimental.pallas import tpu_sc as plsc

_B = 16384
_F = 26
_V = 100000
_D = 16

_NC = 2
_NS = 16
_NW = _NC * _NS
_CHUNK = 128
_NBUF = 2


def _gather_body(nb, embt_hbm, lin16_hbm, xt_hbm, emb_out, lin_out,
                 idx_bufs, idx16_bufs, row_bufs, lin16_bufs,
                 col_bufs, linval_bufs, sems_e, sems_l):
    per_w = nb * _F // _NW
    nsteps = per_w // (_CHUNK * _NBUF)
    cpf = nb // _CHUNK
    wid = lax.axis_index("s") * _NC + lax.axis_index("c")
    base_c = wid * per_w // _CHUNK
    lane_iota = lax.iota(jnp.int32, 16)

    def step(m, carry):
        cs = [base_c + m * _NBUF + b for b in range(_NBUF)]
        fs = [c // cpf for c in cs]
        bs = [(c % cpf) * _CHUNK for c in cs]
        for b in range(_NBUF):
            pltpu.sync_copy(xt_hbm.at[fs[b], pl.ds(bs[b], _CHUNK)],
                            idx_bufs[b])
            for g in range(_CHUNK // 16):
                sl = pl.ds(g * 16, 16)
                idx16_bufs[b][sl] = jnp.right_shift(idx_bufs[b][sl], 4)
        cps = []
        for b in range(_NBUF):
            for d in range(_D):
                cps.append(pltpu.async_copy(
                    embt_hbm.at[fs[b], d].at[idx16_bufs[b]],
                    row_bufs[b][d], sems_e[b]))
            cps.append(pltpu.async_copy(lin16_hbm.at[fs[b], 0].at[idx16_bufs[b]],
                                        lin16_bufs[b], sems_l[b]))
        for c in cps:
            c.wait()
        for b in range(_NBUF):
            lanes = [jnp.bitwise_and(idx_bufs[b][pl.ds(g * 16, 16)], 15)
                     for g in range(_CHUNK // 16)]
            rows = [lane_iota + g * 16 for g in range(_CHUNK // 16)]
            for g in range(_CHUNK // 16):
                sl = pl.ds(g * 16, 16)
                for d in range(_D):
                    col_bufs[b][d, sl] = plsc.load_gather(
                        row_bufs[b][d], [rows[g], lanes[g]])
                linval_bufs[b][sl] = plsc.load_gather(
                    lin16_bufs[b], [rows[g], lanes[g]])
            pltpu.sync_copy(col_bufs[b],
                            emb_out.at[pl.ds(fs[b] * _D, _D),
                                       pl.ds(bs[b], _CHUNK)])
            pltpu.sync_copy(linval_bufs[b],
                            lin_out.at[fs[b], pl.ds(bs[b], _CHUNK)])
        return carry

    lax.fori_loop(0, nsteps, step, 0)


@functools.lru_cache(maxsize=None)
def _make_sc_gather(nb):
    return functools.partial(
        pl.kernel,
        mesh=plsc.VectorSubcoreMesh(core_axis_name="c", subcore_axis_name="s",
                                    num_cores=_NC, num_subcores=_NS),
        out_type=[
            jax.ShapeDtypeStruct((_F * _D, nb), jnp.float32),
            jax.ShapeDtypeStruct((_F, nb), jnp.float32),
        ],
        scratch_types=[
            [pltpu.VMEM((_CHUNK,), jnp.int32) for _ in range(_NBUF)],
            [pltpu.VMEM((_CHUNK,), jnp.int32) for _ in range(_NBUF)],
            [[pltpu.VMEM((_CHUNK, 16), jnp.float32) for _ in range(_D)]
             for _ in range(_NBUF)],
            [pltpu.VMEM((_CHUNK, 16), jnp.float32) for _ in range(_NBUF)],
            [pltpu.VMEM((_D, _CHUNK), jnp.float32) for _ in range(_NBUF)],
            [pltpu.VMEM((_CHUNK,), jnp.float32) for _ in range(_NBUF)],
            [pltpu.SemaphoreType.DMA for _ in range(_NBUF)],
            [pltpu.SemaphoreType.DMA for _ in range(_NBUF)],
        ],
        compiler_params=pltpu.CompilerParams(use_tc_tiling_on_sc=False,
                                             needs_layout_passes=False),
    )(functools.partial(_gather_body, nb))


def _mlp_body(embt_ref, xd_ref, lint_ref, w0e_ref, w0d_ref, b0_ref,
              w1_ref, b1_ref, wo_ref, bo_ref, st_ref, out_ref):
    dotn = functools.partial(jnp.dot, preferred_element_type=jnp.float32,
                             precision=lax.Precision.HIGHEST)

    def dott(a_t, w):
        return lax.dot_general(a_t, w, (((0,), (0,)), ((), ())),
                               preferred_element_type=jnp.float32,
                               precision=lax.Precision.HIGHEST)

    embt = embt_ref[...]
    h = dott(embt, w0e_ref[...])
    h = h + dotn(xd_ref[...], w0d_ref[...])
    h = jnp.maximum(h + b0_ref[...], 0.0)
    h = jnp.maximum(dotn(h, w1_ref[...]) + b1_ref[...], 0.0)
    dnn = dotn(h, wo_ref[...]) + bo_ref[...]
    st = dott(embt, st_ref[...])
    fm = 0.5 * (jnp.sum(st * st, axis=1, keepdims=True)
                - jnp.sum(embt * embt, axis=0)[:, None])
    lin_logit = jnp.sum(lint_ref[...], axis=0)[:, None]
    logit = dnn + fm + lin_logit
    out_ref[...] = 1.0 / (1.0 + jnp.exp(-logit))


def _tc_mlp(embt, x_dense, lint, w0e, w0d, b0, w1, b1, wo, bo, smat):
    nb = embt.shape[1]
    bm = 1024
    grid = (nb // bm,)
    return pl.pallas_call(
        _mlp_body,
        grid=grid,
        in_specs=[
            pl.BlockSpec((_F * _D, bm), lambda i: (0, i)),
            pl.BlockSpec((bm, x_dense.shape[1]), lambda i: (i, 0)),
            pl.BlockSpec((_F, bm), lambda i: (0, i)),
            pl.BlockSpec(w0e.shape, lambda i: (0, 0)),
            pl.BlockSpec(w0d.shape, lambda i: (0, 0)),
            pl.BlockSpec(b0.shape, lambda i: (0,)),
            pl.BlockSpec(w1.shape, lambda i: (0, 0)),
            pl.BlockSpec(b1.shape, lambda i: (0,)),
            pl.BlockSpec(wo.shape, lambda i: (0, 0)),
            pl.BlockSpec(bo.shape, lambda i: (0,)),
            pl.BlockSpec(smat.shape, lambda i: (0, 0)),
        ],
        out_specs=pl.BlockSpec((bm, 1), lambda i: (i, 0)),
        out_shape=jax.ShapeDtypeStruct((nb, 1), jnp.float32),
    )(embt, x_dense, lint, w0e, w0d, b0, w1, b1, wo, bo, smat)


def kernel(x_sparse, x_dense, emb_tables, lin_tables, W0, b0, W1, b1, Wo, bo):
    nf = emb_tables.shape[0]
    v = emb_tables.shape[1]
    d = emb_tables.shape[2]
    xt = x_sparse.astype(jnp.int32).T
    embt = jnp.transpose(emb_tables, (0, 2, 1)).reshape(nf, d, v // 16, 16)
    lin16 = jnp.transpose(lin_tables, (0, 2, 1)).reshape(nf, 1, v // 16, 16)
    w0e = W0[:nf * d]
    w0d = W0[nf * d:]
    smat = jnp.tile(jnp.eye(d, dtype=jnp.float32), (nf, 1))
    hb = _B // 2
    outs = []
    for h in range(2):
        sl = slice(h * hb, (h + 1) * hb)
        embt_out, lin_t = _make_sc_gather(hb)(embt, lin16, xt[:, sl])
        outs.append(_tc_mlp(embt_out, x_dense[sl], lin_t, w0e, w0d, b0,
                            W1, b1, Wo, bo, smat))
    return jnp.concatenate(outs, axis=0).reshape(_B)
